# initial kernel scaffold (unmeasured)
import jax
import jax.numpy as jnp
from jax import lax
from jax.experimental import pallas as pl
from jax.experimental.pallas import tpu as pltpu

N_DEV = 8


def kernel(x, w_mat):
    m, _ = x.shape
    _, n = w_mat.shape
    chunk = m // N_DEV

    def body(x_ref, w_ref, out_ref, comm_ref, send_sems, recv_sems, credit_sem):
        my = lax.axis_index("i")
        left = lax.rem(my + N_DEV - 1, N_DEV)
        right = lax.rem(my + 1, N_DEV)

        out_ref[...] = jnp.dot(
            x_ref[...].astype(jnp.bfloat16),
            w_ref[...].astype(jnp.bfloat16),
            preferred_element_type=jnp.float32,
        )

        barrier_sem = pltpu.get_barrier_semaphore()
        pl.semaphore_signal(barrier_sem, inc=1, device_id=(left,),
                            device_id_type=pl.DeviceIdType.MESH)
        pl.semaphore_signal(barrier_sem, inc=1, device_id=(right,),
                            device_id_type=pl.DeviceIdType.MESH)
        pl.semaphore_wait(barrier_sem, 2)

        for s in range(N_DEV - 1):
            c_send = lax.rem(my + (N_DEV - s), N_DEV)
            c_recv = lax.rem(my + (N_DEV - s - 1), N_DEV)
            rdma = pltpu.make_async_remote_copy(
                src_ref=out_ref.at[pl.ds(c_send * chunk, chunk)],
                dst_ref=comm_ref.at[s],
                send_sem=send_sems.at[s],
                recv_sem=recv_sems.at[s],
                device_id=(right,),
                device_id_type=pl.DeviceIdType.MESH,
            )
            rdma.start()
            rdma.wait()
            pl.semaphore_signal(credit_sem, inc=1, device_id=(left,),
                                device_id_type=pl.DeviceIdType.MESH)
            out_ref[pl.ds(c_recv * chunk, chunk)] += comm_ref[s]


        for s in range(N_DEV - 1):
            c_send = lax.rem(my + 1 + (N_DEV - s), N_DEV)
            rdma = pltpu.make_async_remote_copy(
                src_ref=out_ref.at[pl.ds(c_send * chunk, chunk)],
                dst_ref=out_ref.at[pl.ds(c_send * chunk, chunk)],
                send_sem=send_sems.at[N_DEV - 1 + s],
                recv_sem=recv_sems.at[N_DEV - 1 + s],
                device_id=(right,),
                device_id_type=pl.DeviceIdType.MESH,
            )
            pl.semaphore_wait(credit_sem, 1)
            rdma.start()
            rdma.wait()

    n_sems = 2 * (N_DEV - 1)
    return pl.pallas_call(
        body,
        out_shape=jax.ShapeDtypeStruct((m, n), jnp.float32),
        in_specs=[
            pl.BlockSpec(memory_space=pltpu.VMEM),
            pl.BlockSpec(memory_space=pltpu.VMEM),
        ],
        out_specs=pl.BlockSpec(memory_space=pltpu.VMEM),
        scratch_shapes=[
            pltpu.VMEM((N_DEV - 1, chunk, n), jnp.float32),
            pltpu.SemaphoreType.DMA((n_sems,)),
            pltpu.SemaphoreType.DMA((n_sems,)),
            pltpu.SemaphoreType.REGULAR,
        ],
        compiler_params=pltpu.CompilerParams(collective_id=0),
    )(x, w_mat)


# baseline (device time: 716399 ns/iter reference)
import jax
import jax.numpy as jnp
from jax import lax
from jax.experimental import pallas as pl
from jax.experimental.pallas import tpu as pltpu

N_DEV = 8


def kernel(x, w_mat):
    m, _ = x.shape
    _, n = w_mat.shape
    chunk = m // N_DEV

    def body(x_ref, w_ref, out_ref, comm_ref, send_sems, recv_sems,
             ag_credit_sem, rs_credit_sem):
        my = lax.axis_index("i")
        left = lax.rem(my + N_DEV - 1, N_DEV)
        right = lax.rem(my + 1, N_DEV)

        out_ref[...] = jnp.dot(
            x_ref[...], w_ref[...], preferred_element_type=jnp.float32,
        )

        barrier_sem = pltpu.get_barrier_semaphore()
        pl.semaphore_signal(barrier_sem, inc=1, device_id=(left,),
                            device_id_type=pl.DeviceIdType.MESH)
        pl.semaphore_signal(barrier_sem, inc=1, device_id=(right,),
                            device_id_type=pl.DeviceIdType.MESH)
        pl.semaphore_wait(barrier_sem, 2)

        for s in range(N_DEV - 1):
            c_send = lax.rem(my + (N_DEV - s), N_DEV)
            c_recv = lax.rem(my + (N_DEV - s - 1), N_DEV)
            rdma = pltpu.make_async_remote_copy(
                src_ref=out_ref.at[pl.ds(c_send * chunk, chunk)],
                dst_ref=comm_ref.at[s % 2],
                send_sem=send_sems.at[s],
                recv_sem=recv_sems.at[s],
                device_id=(right,),
                device_id_type=pl.DeviceIdType.MESH,
            )
            if s >= 2:
                pl.semaphore_wait(rs_credit_sem, 1)
            rdma.start()
            rdma.wait()
            pl.semaphore_signal(ag_credit_sem, inc=1, device_id=(left,),
                                device_id_type=pl.DeviceIdType.MESH)
            out_ref[pl.ds(c_recv * chunk, chunk)] += comm_ref[s % 2]
            if s <= N_DEV - 4:
                pl.semaphore_signal(rs_credit_sem, inc=1, device_id=(left,),
                                    device_id_type=pl.DeviceIdType.MESH)


        for s in range(N_DEV - 1):
            c_send = lax.rem(my + 1 + (N_DEV - s), N_DEV)
            rdma = pltpu.make_async_remote_copy(
                src_ref=out_ref.at[pl.ds(c_send * chunk, chunk)],
                dst_ref=out_ref.at[pl.ds(c_send * chunk, chunk)],
                send_sem=send_sems.at[N_DEV - 1 + s],
                recv_sem=recv_sems.at[N_DEV - 1 + s],
                device_id=(right,),
                device_id_type=pl.DeviceIdType.MESH,
            )
            pl.semaphore_wait(ag_credit_sem, 1)
            rdma.start()
            rdma.wait()

    n_sems = 2 * (N_DEV - 1)
    out = pl.pallas_call(
        body,
        out_shape=jax.ShapeDtypeStruct((m, n), jnp.float32),
        in_specs=[
            pl.BlockSpec(memory_space=pltpu.VMEM),
            pl.BlockSpec(memory_space=pltpu.VMEM),
        ],
        out_specs=pl.BlockSpec(memory_space=pltpu.VMEM),
        scratch_shapes=[
            pltpu.VMEM((2, chunk, n), jnp.float32),
            pltpu.SemaphoreType.DMA((n_sems,)),
            pltpu.SemaphoreType.DMA((n_sems,)),
            pltpu.SemaphoreType.REGULAR,
            pltpu.SemaphoreType.REGULAR,
        ],
        compiler_params=pltpu.CompilerParams(
            collective_id=0,
            vmem_limit_bytes=63 * 1024 * 1024,
        ),
    )(x.astype(jnp.bfloat16), w_mat.astype(jnp.bfloat16))
    return out


# device time: 259102 ns/iter; 2.7649x vs baseline; 2.7649x over previous
import jax
import jax.numpy as jnp
from jax import lax
from jax.experimental import pallas as pl
from jax.experimental.pallas import tpu as pltpu

N_DEV = 8


def kernel(x, w_mat):
    m, _ = x.shape
    _, n = w_mat.shape
    chunk = m // N_DEV
    n2 = n // 2

    def body(x_ref, w_ref, out_ref,
             stage_cw, stage_ccw, rs_recv_cw, rs_recv_ccw, ag_cw, ag_ccw,
             send_sems, recv_sems,
             rs_cr_cw, rs_cr_ccw, ag_cr_cw, ag_cr_ccw):
        my = lax.axis_index("i")
        left = lax.rem(my + N_DEV - 1, N_DEV)
        right = lax.rem(my + 1, N_DEV)

        def mod8(v):
            return lax.rem(v, N_DEV)

        def rows(c):
            return pl.ds(c * chunk, chunk)

        out_ref[...] = jnp.dot(
            x_ref[...], w_ref[...], preferred_element_type=jnp.float32,
        )

        barrier_sem = pltpu.get_barrier_semaphore()
        for nbr in (left, right):
            pl.semaphore_signal(barrier_sem, inc=1, device_id=(nbr,),
                                device_id_type=pl.DeviceIdType.MESH)
        pl.semaphore_wait(barrier_sem, 2)

        for s in range(N_DEV - 1):
            sl = s % 2
            c_send_cw = mod8(my + (N_DEV - s))
            c_recv_cw = mod8(my + (N_DEV - s - 1))
            c_send_ccw = mod8(my + s)
            c_recv_ccw = mod8(my + s + 1)

            stage_cw[sl] = out_ref[rows(c_send_cw), :n2].astype(jnp.bfloat16)
            stage_ccw[sl] = out_ref[rows(c_send_ccw), n2:].astype(jnp.bfloat16)

            rdma_cw = pltpu.make_async_remote_copy(
                src_ref=stage_cw.at[sl],
                dst_ref=rs_recv_cw.at[sl],
                send_sem=send_sems.at[s],
                recv_sem=recv_sems.at[s],
                device_id=(right,),
                device_id_type=pl.DeviceIdType.MESH,
            )
            rdma_ccw = pltpu.make_async_remote_copy(
                src_ref=stage_ccw.at[sl],
                dst_ref=rs_recv_ccw.at[sl],
                send_sem=send_sems.at[N_DEV - 1 + s],
                recv_sem=recv_sems.at[N_DEV - 1 + s],
                device_id=(left,),
                device_id_type=pl.DeviceIdType.MESH,
            )
            if s >= 2:
                pl.semaphore_wait(rs_cr_cw, 1)
                pl.semaphore_wait(rs_cr_ccw, 1)
            rdma_cw.start()
            rdma_ccw.start()
            rdma_cw.wait()
            rdma_ccw.wait()

            out_ref[rows(c_recv_cw), :n2] += rs_recv_cw[sl].astype(jnp.float32)
            out_ref[rows(c_recv_ccw), n2:] += rs_recv_ccw[sl].astype(jnp.float32)
            if s <= N_DEV - 4:
                pl.semaphore_signal(rs_cr_cw, inc=1, device_id=(left,),
                                    device_id_type=pl.DeviceIdType.MESH)
                pl.semaphore_signal(rs_cr_ccw, inc=1, device_id=(right,),
                                    device_id_type=pl.DeviceIdType.MESH)


        stage_cw[0] = out_ref[rows(mod8(my + 1)), :n2].astype(jnp.bfloat16)
        stage_ccw[0] = out_ref[rows(mod8(my + N_DEV - 1)), n2:].astype(jnp.bfloat16)

        for s in range(N_DEV - 1):
            sl = s % 2
            src_cw = stage_cw.at[0] if s == 0 else ag_cw.at[(s - 1) % 2]
            src_ccw = stage_ccw.at[0] if s == 0 else ag_ccw.at[(s - 1) % 2]
            rdma_cw = pltpu.make_async_remote_copy(
                src_ref=src_cw,
                dst_ref=ag_cw.at[sl],
                send_sem=send_sems.at[2 * (N_DEV - 1) + s],
                recv_sem=recv_sems.at[2 * (N_DEV - 1) + s],
                device_id=(right,),
                device_id_type=pl.DeviceIdType.MESH,
            )
            rdma_ccw = pltpu.make_async_remote_copy(
                src_ref=src_ccw,
                dst_ref=ag_ccw.at[sl],
                send_sem=send_sems.at[3 * (N_DEV - 1) + s],
                recv_sem=recv_sems.at[3 * (N_DEV - 1) + s],
                device_id=(left,),
                device_id_type=pl.DeviceIdType.MESH,
            )
            if s >= 2:
                pl.semaphore_wait(ag_cr_cw, 1)
                pl.semaphore_wait(ag_cr_ccw, 1)
            rdma_cw.start()
            rdma_ccw.start()
            rdma_cw.wait()
            rdma_ccw.wait()
            if 1 <= s <= N_DEV - 3:
                pl.semaphore_signal(ag_cr_cw, inc=1, device_id=(left,),
                                    device_id_type=pl.DeviceIdType.MESH)
                pl.semaphore_signal(ag_cr_ccw, inc=1, device_id=(right,),
                                    device_id_type=pl.DeviceIdType.MESH)

            c_recv_cw = mod8(my + (N_DEV - s))
            c_recv_ccw = mod8(my + s)
            out_ref[rows(c_recv_cw), :n2] = ag_cw[sl].astype(jnp.float32)
            out_ref[rows(c_recv_ccw), n2:] = ag_ccw[sl].astype(jnp.float32)

    n_sems = 4 * (N_DEV - 1)
    comm = (2, chunk, n2)
    out = pl.pallas_call(
        body,
        out_shape=jax.ShapeDtypeStruct((m, n), jnp.float32),
        in_specs=[
            pl.BlockSpec(memory_space=pltpu.VMEM),
            pl.BlockSpec(memory_space=pltpu.VMEM),
        ],
        out_specs=pl.BlockSpec(memory_space=pltpu.VMEM),
        scratch_shapes=[
            pltpu.VMEM(comm, jnp.bfloat16),
            pltpu.VMEM(comm, jnp.bfloat16),
            pltpu.VMEM(comm, jnp.bfloat16),
            pltpu.VMEM(comm, jnp.bfloat16),
            pltpu.VMEM(comm, jnp.bfloat16),
            pltpu.VMEM(comm, jnp.bfloat16),
            pltpu.SemaphoreType.DMA((n_sems,)),
            pltpu.SemaphoreType.DMA((n_sems,)),
            pltpu.SemaphoreType.REGULAR,
            pltpu.SemaphoreType.REGULAR,
            pltpu.SemaphoreType.REGULAR,
            pltpu.SemaphoreType.REGULAR,
        ],
        compiler_params=pltpu.CompilerParams(
            collective_id=0,
            vmem_limit_bytes=63 * 1024 * 1024,
        ),
    )(x.astype(jnp.bfloat16), w_mat.astype(jnp.bfloat16))
    return out


# device time: 250223 ns/iter; 2.8630x vs baseline; 1.0355x over previous
import jax
import jax.numpy as jnp
from jax import lax
from jax.experimental import pallas as pl
from jax.experimental.pallas import tpu as pltpu

N_DEV = 8


def kernel(x, w_mat):
    m, _ = x.shape
    _, n = w_mat.shape
    chunk = m // N_DEV
    n2 = n // 2

    def body(x_ref, w_ref, out_ref,
             stage_cw, stage_ccw, rs_recv_cw, rs_recv_ccw, ag_cw, ag_ccw,
             send_sems, recv_sems,
             rs_cr_cw, rs_cr_ccw, ag_cr_cw, ag_cr_ccw):
        my = lax.axis_index("i")
        left = lax.rem(my + N_DEV - 1, N_DEV)
        right = lax.rem(my + 1, N_DEV)

        def mod8(v):
            return lax.rem(v, N_DEV)

        def rows(c):
            return pl.ds(c * chunk, chunk)

        def compute_chunk(c):
            out_ref[rows(c), :] = jnp.dot(
                x_ref[rows(c), :], w_ref[...],
                preferred_element_type=jnp.float32,
            )

        def rs_rdma(s, direction):
            stage, rs_recv = (stage_cw, rs_recv_cw) if direction == 0 else (
                stage_ccw, rs_recv_ccw)
            tgt = right if direction == 0 else left
            idx = direction * (N_DEV - 1) + s
            return pltpu.make_async_remote_copy(
                src_ref=stage.at[s % 2],
                dst_ref=rs_recv.at[s % 2],
                send_sem=send_sems.at[idx],
                recv_sem=recv_sems.at[idx],
                device_id=(tgt,),
                device_id_type=pl.DeviceIdType.MESH,
            )

        def ag_rdma(s, direction):
            buf = ag_cw if direction == 0 else ag_ccw
            stage = stage_cw if direction == 0 else stage_ccw
            tgt = right if direction == 0 else left
            idx = (2 + direction) * (N_DEV - 1) + s
            src = stage.at[0] if s == 0 else buf.at[(s - 1) % 2]
            return pltpu.make_async_remote_copy(
                src_ref=src,
                dst_ref=buf.at[s % 2],
                send_sem=send_sems.at[idx],
                recv_sem=recv_sems.at[idx],
                device_id=(tgt,),
                device_id_type=pl.DeviceIdType.MESH,
            )

        barrier_sem = pltpu.get_barrier_semaphore()
        for nbr in (left, right):
            pl.semaphore_signal(barrier_sem, inc=1, device_id=(nbr,),
                                device_id_type=pl.DeviceIdType.MESH)
        pl.semaphore_wait(barrier_sem, 2)

        compute_chunk(my)
        stage_cw[0] = out_ref[rows(my), :n2].astype(jnp.bfloat16)
        stage_ccw[0] = out_ref[rows(my), n2:].astype(jnp.bfloat16)
        rs0_cw = rs_rdma(0, 0)
        rs0_ccw = rs_rdma(0, 1)
        rs0_cw.start()
        rs0_ccw.start()
        for d in range(1, N_DEV):
            compute_chunk(mod8(my + d))

        pending = (rs0_cw, rs0_ccw)
        for s in range(N_DEV - 1):
            sl = s % 2
            pending[0].wait()
            pending[1].wait()
            c_recv_cw = mod8(my + (N_DEV - s - 1))
            c_recv_ccw = mod8(my + s + 1)
            t_cw = (out_ref[rows(c_recv_cw), :n2]
                    + rs_recv_cw[sl].astype(jnp.float32))
            out_ref[rows(c_recv_cw), :n2] = t_cw
            t_ccw = (out_ref[rows(c_recv_ccw), n2:]
                     + rs_recv_ccw[sl].astype(jnp.float32))
            out_ref[rows(c_recv_ccw), n2:] = t_ccw
            if s < N_DEV - 2:
                nsl = (s + 1) % 2
                stage_cw[nsl] = t_cw.astype(jnp.bfloat16)
                stage_ccw[nsl] = t_ccw.astype(jnp.bfloat16)
            if s <= N_DEV - 4:
                pl.semaphore_signal(rs_cr_cw, inc=1, device_id=(left,),
                                    device_id_type=pl.DeviceIdType.MESH)
                pl.semaphore_signal(rs_cr_ccw, inc=1, device_id=(right,),
                                    device_id_type=pl.DeviceIdType.MESH)
            if s < N_DEV - 2:
                if s >= 1:
                    pl.semaphore_wait(rs_cr_cw, 1)
                    pl.semaphore_wait(rs_cr_ccw, 1)
                nxt_cw = rs_rdma(s + 1, 0)
                nxt_ccw = rs_rdma(s + 1, 1)
                nxt_cw.start()
                nxt_ccw.start()
                pending = (nxt_cw, nxt_ccw)

        stage_cw[0] = out_ref[rows(mod8(my + 1)), :n2].astype(jnp.bfloat16)
        stage_ccw[0] = out_ref[rows(mod8(my + N_DEV - 1)), n2:].astype(jnp.bfloat16)

        ag0_cw = ag_rdma(0, 0)
        ag0_ccw = ag_rdma(0, 1)
        ag0_cw.start()
        ag0_ccw.start()
        pending = (ag0_cw, ag0_ccw)
        for s in range(N_DEV - 1):
            sl = s % 2
            pending[0].wait()
            pending[1].wait()
            if 1 <= s <= N_DEV - 3:
                pl.semaphore_signal(ag_cr_cw, inc=1, device_id=(left,),
                                    device_id_type=pl.DeviceIdType.MESH)
                pl.semaphore_signal(ag_cr_ccw, inc=1, device_id=(right,),
                                    device_id_type=pl.DeviceIdType.MESH)
            if s < N_DEV - 2:
                if s >= 1:
                    pl.semaphore_wait(ag_cr_cw, 1)
                    pl.semaphore_wait(ag_cr_ccw, 1)
                nxt_cw = ag_rdma(s + 1, 0)
                nxt_ccw = ag_rdma(s + 1, 1)
                nxt_cw.start()
                nxt_ccw.start()
                pending = (nxt_cw, nxt_ccw)
            c_recv_cw = mod8(my + (N_DEV - s))
            c_recv_ccw = mod8(my + s)
            out_ref[rows(c_recv_cw), :n2] = ag_cw[sl].astype(jnp.float32)
            out_ref[rows(c_recv_ccw), n2:] = ag_ccw[sl].astype(jnp.float32)

    n_sems = 4 * (N_DEV - 1)
    comm = (2, chunk, n2)
    out = pl.pallas_call(
        body,
        out_shape=jax.ShapeDtypeStruct((m, n), jnp.float32),
        in_specs=[
            pl.BlockSpec(memory_space=pltpu.VMEM),
            pl.BlockSpec(memory_space=pltpu.VMEM),
        ],
        out_specs=pl.BlockSpec(memory_space=pltpu.VMEM),
        scratch_shapes=[
            pltpu.VMEM(comm, jnp.bfloat16),
            pltpu.VMEM(comm, jnp.bfloat16),
            pltpu.VMEM(comm, jnp.bfloat16),
            pltpu.VMEM(comm, jnp.bfloat16),
            pltpu.VMEM(comm, jnp.bfloat16),
            pltpu.VMEM(comm, jnp.bfloat16),
            pltpu.SemaphoreType.DMA((n_sems,)),
            pltpu.SemaphoreType.DMA((n_sems,)),
            pltpu.SemaphoreType.REGULAR,
            pltpu.SemaphoreType.REGULAR,
            pltpu.SemaphoreType.REGULAR,
            pltpu.SemaphoreType.REGULAR,
        ],
        compiler_params=pltpu.CompilerParams(
            collective_id=0,
            vmem_limit_bytes=63 * 1024 * 1024,
        ),
    )(x.astype(jnp.bfloat16), w_mat.astype(jnp.bfloat16))
    return out


# device time: 215303 ns/iter; 3.3274x vs baseline; 1.1622x over previous
import jax
import jax.numpy as jnp
from jax import lax
from jax.experimental import pallas as pl
from jax.experimental.pallas import tpu as pltpu

N_DEV = 8
N_RING = 4


def kernel(x, w_mat):
    m, _ = x.shape
    _, n = w_mat.shape
    chunk = m // N_DEV
    nq = n // N_RING

    RINGS = ((0, 0), (1, 0), (0, 1), (1, 1))

    def body(x_ref, w_ref, out_ref, stage, rs_recv, ag_buf,
             send_sems, recv_sems,
             rs_cr_0, rs_cr_1, rs_cr_2, rs_cr_3,
             ag_cr_0, ag_cr_1, ag_cr_2, ag_cr_3):
        my = lax.axis_index("i")
        left = lax.rem(my + N_DEV - 1, N_DEV)
        right = lax.rem(my + 1, N_DEV)
        rs_cr = (rs_cr_0, rs_cr_1, rs_cr_2, rs_cr_3)
        ag_cr = (ag_cr_0, ag_cr_1, ag_cr_2, ag_cr_3)

        def mod8(v):
            return lax.rem(v, N_DEV)

        def rows(c):
            return pl.ds(c * chunk, chunk)

        def cols(r):
            d, q = RINGS[r]
            return slice((2 * d + q) * nq, (2 * d + q + 1) * nq)

        def tgt(r):
            return right if RINGS[r][0] == 0 else left

        def writer(r):
            return left if RINGS[r][0] == 0 else right

        def c_send_rs(r, s):
            return mod8(my + (N_DEV - s) if RINGS[r][0] == 0 else my + s)

        def c_recv_rs(r, s):
            return mod8(my + (N_DEV - s - 1) if RINGS[r][0] == 0
                        else my + s + 1)

        def c_recv_ag(r, s):
            return mod8(my + (N_DEV - s) if RINGS[r][0] == 0 else my + s)

        def compute_chunk(c):
            out_ref[rows(c), :] = jnp.dot(
                x_ref[rows(c), :], w_ref[...],
                preferred_element_type=jnp.float32,
            )

        def rs_rdma(r, s):
            idx = r * (N_DEV - 1) + s
            return pltpu.make_async_remote_copy(
                src_ref=stage.at[r, s % 2],
                dst_ref=rs_recv.at[r, s % 2],
                send_sem=send_sems.at[idx],
                recv_sem=recv_sems.at[idx],
                device_id=(tgt(r),),
                device_id_type=pl.DeviceIdType.MESH,
            )

        def ag_rdma(r, s):
            idx = (N_RING + r) * (N_DEV - 1) + s
            src = stage.at[r, 0] if s == 0 else ag_buf.at[r, (s - 1) % 2]
            return pltpu.make_async_remote_copy(
                src_ref=src,
                dst_ref=ag_buf.at[r, s % 2],
                send_sem=send_sems.at[idx],
                recv_sem=recv_sems.at[idx],
                device_id=(tgt(r),),
                device_id_type=pl.DeviceIdType.MESH,
            )

        def signal(sem, to):
            pl.semaphore_signal(sem, inc=1, device_id=(to,),
                                device_id_type=pl.DeviceIdType.MESH)

        barrier_sem = pltpu.get_barrier_semaphore()
        signal(barrier_sem, left)
        signal(barrier_sem, right)
        pl.semaphore_wait(barrier_sem, 2)

        compute_chunk(my)
        for r in range(N_RING):
            stage[r, 0] = out_ref[rows(my), cols(r)].astype(jnp.bfloat16)
        pending = [rs_rdma(r, 0) for r in range(N_RING)]
        for p in pending:
            p.start()
        for d in (1, 7, 2, 6, 3, 5, 4):
            compute_chunk(mod8(my + d))

        for s in range(N_DEV - 1):
            sl = s % 2
            for r in range(N_RING):
                if r == 0 or r == 2:
                    pending[r].wait()
                    pending[r + 1].wait()
                c = c_recv_rs(r, s)
                t = (out_ref[rows(c), cols(r)]
                     + rs_recv[r, sl].astype(jnp.float32))
                out_ref[rows(c), cols(r)] = t
                if s < N_DEV - 2:
                    stage[r, (s + 1) % 2] = t.astype(jnp.bfloat16)
                if s <= N_DEV - 4:
                    signal(rs_cr[r], writer(r))
                if r == 1 or r == 3:
                    if s < N_DEV - 2:
                        for rr in (r - 1, r):
                            if s >= 1:
                                pl.semaphore_wait(rs_cr[rr], 1)
                            nxt = rs_rdma(rr, s + 1)
                            nxt.start()
                            pending[rr] = nxt

        for r in range(N_RING):
            own = mod8(my + 1) if RINGS[r][0] == 0 else mod8(my + N_DEV - 1)
            stage[r, 0] = out_ref[rows(own), cols(r)].astype(jnp.bfloat16)

        pending = [ag_rdma(r, 0) for r in range(N_RING)]
        for p in pending:
            p.start()
        for s in range(N_DEV - 1):
            sl = s % 2
            for g in (0, 2):
                pending[g].wait()
                pending[g + 1].wait()
                for r in (g, g + 1):
                    if 1 <= s <= N_DEV - 3:
                        signal(ag_cr[r], writer(r))
                if s < N_DEV - 2:
                    for r in (g, g + 1):
                        if s >= 1:
                            pl.semaphore_wait(ag_cr[r], 1)
                        nxt = ag_rdma(r, s + 1)
                        nxt.start()
                        pending[r] = nxt
                for r in (g, g + 1):
                    out_ref[rows(c_recv_ag(r, s)), cols(r)] = (
                        ag_buf[r, sl].astype(jnp.float32))

    n_sems = 2 * N_RING * (N_DEV - 1)
    comm = (N_RING, 2, chunk, nq)
    out = pl.pallas_call(
        body,
        out_shape=jax.ShapeDtypeStruct((m, n), jnp.float32),
        in_specs=[
            pl.BlockSpec(memory_space=pltpu.VMEM),
            pl.BlockSpec(memory_space=pltpu.VMEM),
        ],
        out_specs=pl.BlockSpec(memory_space=pltpu.VMEM),
        scratch_shapes=[
            pltpu.VMEM(comm, jnp.bfloat16),
            pltpu.VMEM(comm, jnp.bfloat16),
            pltpu.VMEM(comm, jnp.bfloat16),
            pltpu.SemaphoreType.DMA((n_sems,)),
            pltpu.SemaphoreType.DMA((n_sems,)),
        ] + [pltpu.SemaphoreType.REGULAR] * 8,
        compiler_params=pltpu.CompilerParams(
            collective_id=0,
            vmem_limit_bytes=63 * 1024 * 1024,
        ),
    )(x.astype(jnp.bfloat16), w_mat.astype(jnp.bfloat16))
    return out


# device time: 215093 ns/iter; 3.3306x vs baseline; 1.0010x over previous
import jax
import jax.numpy as jnp
from jax import lax
from jax.experimental import pallas as pl
from jax.experimental.pallas import tpu as pltpu

N_DEV = 8
N_RING = 4


def kernel(x, w_mat):
    m, _ = x.shape
    _, n = w_mat.shape
    chunk = m // N_DEV
    nq = n // N_RING

    RINGS = ((0, 0), (1, 0), (0, 1), (1, 1))

    def body(x_ref, w_ref, out_ref, stage, rs_recv, ag_buf,
             send_sems, recv_sems,
             rs_cr_0, rs_cr_1, rs_cr_2, rs_cr_3,
             ag_cr_0, ag_cr_1, ag_cr_2, ag_cr_3):
        my = lax.axis_index("i")
        left = lax.rem(my + N_DEV - 1, N_DEV)
        right = lax.rem(my + 1, N_DEV)
        rs_cr = (rs_cr_0, rs_cr_1, rs_cr_2, rs_cr_3)
        ag_cr = (ag_cr_0, ag_cr_1, ag_cr_2, ag_cr_3)

        def mod8(v):
            return lax.rem(v, N_DEV)

        def rows(c):
            return pl.ds(c * chunk, chunk)

        def cols(r):
            d, q = RINGS[r]
            return slice((2 * d + q) * nq, (2 * d + q + 1) * nq)

        def tgt(r):
            return right if RINGS[r][0] == 0 else left

        def writer(r):
            return left if RINGS[r][0] == 0 else right

        def c_send_rs(r, s):
            return mod8(my + (N_DEV - s) if RINGS[r][0] == 0 else my + s)

        def c_recv_rs(r, s):
            return mod8(my + (N_DEV - s - 1) if RINGS[r][0] == 0
                        else my + s + 1)

        def c_recv_ag(r, s):
            return mod8(my + (N_DEV - s) if RINGS[r][0] == 0 else my + s)

        def compute_chunk(c):
            out_ref[rows(c), :] = jnp.dot(
                x_ref[rows(c), :], w_ref[...],
                preferred_element_type=jnp.float32,
            )

        def rs_rdma(r, s):
            idx = r * (N_DEV - 1) + s
            return pltpu.make_async_remote_copy(
                src_ref=stage.at[r, s % 2],
                dst_ref=rs_recv.at[r, s % 2],
                send_sem=send_sems.at[idx],
                recv_sem=recv_sems.at[idx],
                device_id=(tgt(r),),
                device_id_type=pl.DeviceIdType.MESH,
            )

        def ag_rdma(r, s):
            idx = (N_RING + r) * (N_DEV - 1) + s
            src = stage.at[r, 0] if s == 0 else ag_buf.at[r, (s - 1) % 2]
            return pltpu.make_async_remote_copy(
                src_ref=src,
                dst_ref=ag_buf.at[r, s % 2],
                send_sem=send_sems.at[idx],
                recv_sem=recv_sems.at[idx],
                device_id=(tgt(r),),
                device_id_type=pl.DeviceIdType.MESH,
            )

        def signal(sem, to):
            pl.semaphore_signal(sem, inc=1, device_id=(to,),
                                device_id_type=pl.DeviceIdType.MESH)

        barrier_sem = pltpu.get_barrier_semaphore()
        signal(barrier_sem, left)
        signal(barrier_sem, right)
        pl.semaphore_wait(barrier_sem, 2)

        compute_chunk(my)
        for r in range(N_RING):
            stage[r, 0] = out_ref[rows(my), cols(r)].astype(jnp.bfloat16)
        pending = [rs_rdma(r, 0) for r in range(N_RING)]
        for p in pending:
            p.start()
        for d in (1, 7, 2, 6, 3, 5, 4):
            compute_chunk(mod8(my + d))

        for s in range(N_DEV - 1):
            sl = s % 2
            for r in range(N_RING):
                if r == 0 or r == 2:
                    pending[r].wait()
                    pending[r + 1].wait()
                c = c_recv_rs(r, s)
                t = (out_ref[rows(c), cols(r)]
                     + rs_recv[r, sl].astype(jnp.float32))
                if s < N_DEV - 2:
                    stage[r, (s + 1) % 2] = t.astype(jnp.bfloat16)
                else:
                    out_ref[rows(c), cols(r)] = t
                    stage[r, 0] = t.astype(jnp.bfloat16)
                if s <= N_DEV - 4:
                    signal(rs_cr[r], writer(r))
                if r == 1 or r == 3:
                    if s < N_DEV - 2:
                        for rr in (r - 1, r):
                            if s >= 1:
                                pl.semaphore_wait(rs_cr[rr], 1)
                            nxt = rs_rdma(rr, s + 1)
                            nxt.start()
                            pending[rr] = nxt

        pending = [ag_rdma(r, 0) for r in range(N_RING)]
        for p in pending:
            p.start()
        for s in range(N_DEV - 1):
            sl = s % 2
            for g in (0, 2):
                pending[g].wait()
                pending[g + 1].wait()
                for r in (g, g + 1):
                    if 1 <= s <= N_DEV - 3:
                        signal(ag_cr[r], writer(r))
                if s < N_DEV - 2:
                    for r in (g, g + 1):
                        if s >= 1:
                            pl.semaphore_wait(ag_cr[r], 1)
                        nxt = ag_rdma(r, s + 1)
                        nxt.start()
                        pending[r] = nxt
                for r in (g, g + 1):
                    out_ref[rows(c_recv_ag(r, s)), cols(r)] = (
                        ag_buf[r, sl].astype(jnp.float32))

    n_sems = 2 * N_RING * (N_DEV - 1)
    comm = (N_RING, 2, chunk, nq)
    out = pl.pallas_call(
        body,
        out_shape=jax.ShapeDtypeStruct((m, n), jnp.float32),
        in_specs=[
            pl.BlockSpec(memory_space=pltpu.VMEM),
            pl.BlockSpec(memory_space=pltpu.VMEM),
        ],
        out_specs=pl.BlockSpec(memory_space=pltpu.VMEM),
        scratch_shapes=[
            pltpu.VMEM(comm, jnp.bfloat16),
            pltpu.VMEM(comm, jnp.bfloat16),
            pltpu.VMEM(comm, jnp.bfloat16),
            pltpu.SemaphoreType.DMA((n_sems,)),
            pltpu.SemaphoreType.DMA((n_sems,)),
        ] + [pltpu.SemaphoreType.REGULAR] * 8,
        compiler_params=pltpu.CompilerParams(
            collective_id=0,
            vmem_limit_bytes=63 * 1024 * 1024,
        ),
    )(x.astype(jnp.bfloat16), w_mat.astype(jnp.bfloat16))
    return out


# device time: 207592 ns/iter; 3.4510x vs baseline; 1.0361x over previous
import jax
import jax.numpy as jnp
from jax import lax
from jax.experimental import pallas as pl
from jax.experimental.pallas import tpu as pltpu

N_DEV = 8
N_RING = 4


def kernel(x, w_mat):
    m, _ = x.shape
    k, n = w_mat.shape
    chunk = m // N_DEV
    nq = n // N_RING

    RINGS = ((0, 0), (1, 0), (0, 1), (1, 1))

    def body(x_ref, w_ref, out_ref, w_bf, stage, rs_recv, ag_buf,
             send_sems, recv_sems,
             rs_cr_0, rs_cr_1, rs_cr_2, rs_cr_3,
             ag_cr_0, ag_cr_1, ag_cr_2, ag_cr_3):
        my = lax.axis_index("i")
        left = lax.rem(my + N_DEV - 1, N_DEV)
        right = lax.rem(my + 1, N_DEV)
        rs_cr = (rs_cr_0, rs_cr_1, rs_cr_2, rs_cr_3)
        ag_cr = (ag_cr_0, ag_cr_1, ag_cr_2, ag_cr_3)

        def mod8(v):
            return lax.rem(v, N_DEV)

        def rows(c):
            return pl.ds(c * chunk, chunk)

        def cols(r):
            d, q = RINGS[r]
            return slice((2 * d + q) * nq, (2 * d + q + 1) * nq)

        def tgt(r):
            return right if RINGS[r][0] == 0 else left

        def writer(r):
            return left if RINGS[r][0] == 0 else right

        def c_send_rs(r, s):
            return mod8(my + (N_DEV - s) if RINGS[r][0] == 0 else my + s)

        def c_recv_rs(r, s):
            return mod8(my + (N_DEV - s - 1) if RINGS[r][0] == 0
                        else my + s + 1)

        def c_recv_ag(r, s):
            return mod8(my + (N_DEV - s) if RINGS[r][0] == 0 else my + s)

        def compute_chunk(c):
            out_ref[rows(c), :] = jnp.dot(
                x_ref[rows(c), :].astype(jnp.bfloat16), w_bf[...],
                preferred_element_type=jnp.float32,
            )

        def rs_rdma(r, s):
            idx = r * (N_DEV - 1) + s
            return pltpu.make_async_remote_copy(
                src_ref=stage.at[r, s % 2],
                dst_ref=rs_recv.at[r, s % 2],
                send_sem=send_sems.at[idx],
                recv_sem=recv_sems.at[idx],
                device_id=(tgt(r),),
                device_id_type=pl.DeviceIdType.MESH,
            )

        def ag_rdma(r, s):
            idx = (N_RING + r) * (N_DEV - 1) + s
            src = stage.at[r, 0] if s == 0 else ag_buf.at[r, (s - 1) % 2]
            return pltpu.make_async_remote_copy(
                src_ref=src,
                dst_ref=ag_buf.at[r, s % 2],
                send_sem=send_sems.at[idx],
                recv_sem=recv_sems.at[idx],
                device_id=(tgt(r),),
                device_id_type=pl.DeviceIdType.MESH,
            )

        def signal(sem, to):
            pl.semaphore_signal(sem, inc=1, device_id=(to,),
                                device_id_type=pl.DeviceIdType.MESH)

        barrier_sem = pltpu.get_barrier_semaphore()
        signal(barrier_sem, left)
        signal(barrier_sem, right)
        pl.semaphore_wait(barrier_sem, 2)

        w_bf[...] = w_ref[...].astype(jnp.bfloat16)
        compute_chunk(my)
        for r in range(N_RING):
            stage[r, 0] = out_ref[rows(my), cols(r)].astype(jnp.bfloat16)
        pending = [rs_rdma(r, 0) for r in range(N_RING)]
        for p in pending:
            p.start()
        for d in (1, 7, 2, 6, 3, 5, 4):
            compute_chunk(mod8(my + d))

        for s in range(N_DEV - 1):
            sl = s % 2
            for r in range(N_RING):
                if r == 0 or r == 2:
                    pending[r].wait()
                    pending[r + 1].wait()
                c = c_recv_rs(r, s)
                t = (out_ref[rows(c), cols(r)]
                     + rs_recv[r, sl].astype(jnp.float32))
                if s < N_DEV - 2:
                    stage[r, (s + 1) % 2] = t.astype(jnp.bfloat16)
                else:
                    out_ref[rows(c), cols(r)] = t
                    stage[r, 0] = t.astype(jnp.bfloat16)
                if s <= N_DEV - 4:
                    signal(rs_cr[r], writer(r))
                if r == 1 or r == 3:
                    if s < N_DEV - 2:
                        for rr in (r - 1, r):
                            if s >= 1:
                                pl.semaphore_wait(rs_cr[rr], 1)
                            nxt = rs_rdma(rr, s + 1)
                            nxt.start()
                            pending[rr] = nxt

        pending = [ag_rdma(r, 0) for r in range(N_RING)]
        for p in pending:
            p.start()
        for s in range(N_DEV - 1):
            sl = s % 2
            for g in (0, 2):
                pending[g].wait()
                pending[g + 1].wait()
                for r in (g, g + 1):
                    if 1 <= s <= N_DEV - 3:
                        signal(ag_cr[r], writer(r))
                if s < N_DEV - 2:
                    for r in (g, g + 1):
                        if s >= 1:
                            pl.semaphore_wait(ag_cr[r], 1)
                        nxt = ag_rdma(r, s + 1)
                        nxt.start()
                        pending[r] = nxt
                for r in (g, g + 1):
                    out_ref[rows(c_recv_ag(r, s)), cols(r)] = (
                        ag_buf[r, sl].astype(jnp.float32))

    n_sems = 2 * N_RING * (N_DEV - 1)
    comm = (N_RING, 2, chunk, nq)
    out = pl.pallas_call(
        body,
        out_shape=jax.ShapeDtypeStruct((m, n), jnp.float32),
        in_specs=[
            pl.BlockSpec(memory_space=pltpu.VMEM),
            pl.BlockSpec(memory_space=pltpu.VMEM),
        ],
        out_specs=pl.BlockSpec(memory_space=pltpu.VMEM),
        scratch_shapes=[
            pltpu.VMEM((k, n), jnp.bfloat16),
            pltpu.VMEM(comm, jnp.bfloat16),
            pltpu.VMEM(comm, jnp.bfloat16),
            pltpu.VMEM(comm, jnp.bfloat16),
            pltpu.SemaphoreType.DMA((n_sems,)),
            pltpu.SemaphoreType.DMA((n_sems,)),
        ] + [pltpu.SemaphoreType.REGULAR] * 8,
        compiler_params=pltpu.CompilerParams(
            collective_id=0,
            vmem_limit_bytes=63 * 1024 * 1024,
        ),
    )(x, w_mat)
    return out


# device time: 203893 ns/iter; 3.5136x vs baseline; 1.0181x over previous
import jax
import jax.numpy as jnp
from jax import lax
from jax.experimental import pallas as pl
from jax.experimental.pallas import tpu as pltpu

N_DEV = 8
N_RING = 4


def kernel(x, w_mat):
    m, _ = x.shape
    k, n = w_mat.shape
    chunk = m // N_DEV
    nq = n // N_RING

    RINGS = ((0, 0), (1, 0), (0, 1), (1, 1))

    def body(x_ref, w_ref, out_ref, w_bf, stage, rs_recv, ag_buf,
             send_sems, recv_sems,
             rs_cr_0, rs_cr_1, rs_cr_2, rs_cr_3,
             ag_cr_0, ag_cr_1, ag_cr_2, ag_cr_3):
        my = lax.axis_index("i")
        left = lax.rem(my + N_DEV - 1, N_DEV)
        right = lax.rem(my + 1, N_DEV)
        rs_cr = (rs_cr_0, rs_cr_1, rs_cr_2, rs_cr_3)
        ag_cr = (ag_cr_0, ag_cr_1, ag_cr_2, ag_cr_3)

        def mod8(v):
            return lax.rem(v, N_DEV)

        def rows(c):
            return pl.ds(c * chunk, chunk)

        def cols(r):
            d, q = RINGS[r]
            return slice((2 * d + q) * nq, (2 * d + q + 1) * nq)

        def tgt(r):
            return right if RINGS[r][0] == 0 else left

        def writer(r):
            return left if RINGS[r][0] == 0 else right

        def c_send_rs(r, s):
            return mod8(my + (N_DEV - s) if RINGS[r][0] == 0 else my + s)

        def c_recv_rs(r, s):
            return mod8(my + (N_DEV - s - 1) if RINGS[r][0] == 0
                        else my + s + 1)

        def c_recv_ag(r, s):
            return mod8(my + (N_DEV - s) if RINGS[r][0] == 0 else my + s)

        def compute_chunk(c):
            out_ref[rows(c), :] = jnp.dot(
                x_ref[rows(c), :].astype(jnp.bfloat16), w_bf[...],
                preferred_element_type=jnp.float32,
            )

        def rs_rdma(r, s):
            idx = r * (N_DEV - 1) + s
            return pltpu.make_async_remote_copy(
                src_ref=stage.at[r, s % 2],
                dst_ref=rs_recv.at[r, s % 2],
                send_sem=send_sems.at[idx],
                recv_sem=recv_sems.at[idx],
                device_id=(tgt(r),),
                device_id_type=pl.DeviceIdType.MESH,
            )

        def ag_rdma(r, s):
            idx = (N_RING + r) * (N_DEV - 1) + s
            src = stage.at[r, 0] if s == 0 else ag_buf.at[r, (s - 1) % 2]
            return pltpu.make_async_remote_copy(
                src_ref=src,
                dst_ref=ag_buf.at[r, s % 2],
                send_sem=send_sems.at[idx],
                recv_sem=recv_sems.at[idx],
                device_id=(tgt(r),),
                device_id_type=pl.DeviceIdType.MESH,
            )

        def signal(sem, to):
            pl.semaphore_signal(sem, inc=1, device_id=(to,),
                                device_id_type=pl.DeviceIdType.MESH)

        barrier_sem = pltpu.get_barrier_semaphore()
        signal(barrier_sem, left)
        signal(barrier_sem, right)

        w_bf[...] = w_ref[...].astype(jnp.bfloat16)
        compute_chunk(my)
        for r in range(N_RING):
            stage[r, 0] = out_ref[rows(my), cols(r)].astype(jnp.bfloat16)
        pl.semaphore_wait(barrier_sem, 2)
        pending = [rs_rdma(r, 0) for r in range(N_RING)]
        for p in pending:
            p.start()
        for d in (1, 7, 2, 6, 3, 5, 4):
            compute_chunk(mod8(my + d))

        for s in range(N_DEV - 2):
            sl = s % 2
            for r in range(N_RING):
                if r == 0 or r == 2:
                    pending[r].wait()
                    pending[r + 1].wait()
                c = c_recv_rs(r, s)
                t = (out_ref[rows(c), cols(r)]
                     + rs_recv[r, sl].astype(jnp.float32))
                stage[r, (s + 1) % 2] = t.astype(jnp.bfloat16)
                if s <= N_DEV - 4:
                    signal(rs_cr[r], writer(r))
                if r == 1 or r == 3:
                    for rr in (r - 1, r):
                        if s >= 1:
                            pl.semaphore_wait(rs_cr[rr], 1)
                        nxt = rs_rdma(rr, s + 1)
                        nxt.start()
                        pending[rr] = nxt

        for g in (0, 2):
            pending[g].wait()
            pending[g + 1].wait()
            ts = {}
            for r in (g, g + 1):
                c = c_recv_rs(r, N_DEV - 2)
                t = (out_ref[rows(c), cols(r)]
                     + rs_recv[r, (N_DEV - 2) % 2].astype(jnp.float32))
                stage[r, 0] = t.astype(jnp.bfloat16)
                ts[r] = (c, t)
            for r in (g, g + 1):
                nxt = ag_rdma(r, 0)
                nxt.start()
                pending[r] = nxt
            for r in (g, g + 1):
                c, t = ts[r]
                out_ref[rows(c), cols(r)] = t

        for s in range(N_DEV - 1):
            sl = s % 2
            for g in (0, 2):
                pending[g].wait()
                pending[g + 1].wait()
                for r in (g, g + 1):
                    if 1 <= s <= N_DEV - 3:
                        signal(ag_cr[r], writer(r))
                if s < N_DEV - 2:
                    for r in (g, g + 1):
                        if s >= 1:
                            pl.semaphore_wait(ag_cr[r], 1)
                        nxt = ag_rdma(r, s + 1)
                        nxt.start()
                        pending[r] = nxt
                for r in (g, g + 1):
                    out_ref[rows(c_recv_ag(r, s)), cols(r)] = (
                        ag_buf[r, sl].astype(jnp.float32))

    n_sems = 2 * N_RING * (N_DEV - 1)
    comm = (N_RING, 2, chunk, nq)
    out = pl.pallas_call(
        body,
        out_shape=jax.ShapeDtypeStruct((m, n), jnp.float32),
        in_specs=[
            pl.BlockSpec(memory_space=pltpu.VMEM),
            pl.BlockSpec(memory_space=pltpu.VMEM),
        ],
        out_specs=pl.BlockSpec(memory_space=pltpu.VMEM),
        scratch_shapes=[
            pltpu.VMEM((k, n), jnp.bfloat16),
            pltpu.VMEM(comm, jnp.bfloat16),
            pltpu.VMEM(comm, jnp.bfloat16),
            pltpu.VMEM(comm, jnp.bfloat16),
            pltpu.SemaphoreType.DMA((n_sems,)),
            pltpu.SemaphoreType.DMA((n_sems,)),
        ] + [pltpu.SemaphoreType.REGULAR] * 8,
        compiler_params=pltpu.CompilerParams(
            collective_id=0,
            vmem_limit_bytes=63 * 1024 * 1024,
        ),
    )(x, w_mat)
    return out


# device time: 203629 ns/iter; 3.5182x vs baseline; 1.0013x over previous
import jax
import jax.numpy as jnp
from jax import lax
from jax.experimental import pallas as pl
from jax.experimental.pallas import tpu as pltpu

N_DEV = 8
N_PER_DIR = 4
N_RING = 2 * N_PER_DIR
GROUPS = tuple((2 * q, 2 * q + 1) for q in range(N_PER_DIR))


def kernel(x, w_mat):
    m, _ = x.shape
    k, n = w_mat.shape
    chunk = m // N_DEV
    nq = n // N_RING

    def body(x_ref, w_ref, out_ref, w_bf, stage, rs_recv, ag_buf,
             send_sems, recv_sems, *credit_sems):
        my = lax.axis_index("i")
        left = lax.rem(my + N_DEV - 1, N_DEV)
        right = lax.rem(my + 1, N_DEV)
        rs_cr = credit_sems[:N_RING]
        ag_cr = credit_sems[N_RING:]

        def mod8(v):
            return lax.rem(v, N_DEV)

        def rows(c):
            return pl.ds(c * chunk, chunk)

        def cols(r):
            d, q = r % 2, r // 2
            return slice((d * N_PER_DIR + q) * nq,
                         (d * N_PER_DIR + q + 1) * nq)

        def tgt(r):
            return right if r % 2 == 0 else left

        def writer(r):
            return left if r % 2 == 0 else right

        def c_send_rs(r, s):
            return mod8(my + (N_DEV - s) if r % 2 == 0 else my + s)

        def c_recv_rs(r, s):
            return mod8(my + (N_DEV - s - 1) if r % 2 == 0 else my + s + 1)

        def c_recv_ag(r, s):
            return mod8(my + (N_DEV - s) if r % 2 == 0 else my + s)

        def compute_chunk(c):
            out_ref[rows(c), :] = jnp.dot(
                x_ref[rows(c), :].astype(jnp.bfloat16), w_bf[...],
                preferred_element_type=jnp.float32,
            )

        def rs_rdma(r, s):
            idx = r * 2 + s % 2
            return pltpu.make_async_remote_copy(
                src_ref=stage.at[r, s % 2],
                dst_ref=rs_recv.at[r, s % 2],
                send_sem=send_sems.at[idx],
                recv_sem=recv_sems.at[idx],
                device_id=(tgt(r),),
                device_id_type=pl.DeviceIdType.MESH,
            )

        def ag_rdma(r, s):
            idx = 2 * N_RING + r * 2 + s % 2
            src = stage.at[r, 0] if s == 0 else ag_buf.at[r, (s - 1) % 2]
            return pltpu.make_async_remote_copy(
                src_ref=src,
                dst_ref=ag_buf.at[r, s % 2],
                send_sem=send_sems.at[idx],
                recv_sem=recv_sems.at[idx],
                device_id=(tgt(r),),
                device_id_type=pl.DeviceIdType.MESH,
            )

        def signal(sem, to):
            pl.semaphore_signal(sem, inc=1, device_id=(to,),
                                device_id_type=pl.DeviceIdType.MESH)

        barrier_sem = pltpu.get_barrier_semaphore()
        signal(barrier_sem, left)
        signal(barrier_sem, right)

        w_bf[...] = w_ref[...].astype(jnp.bfloat16)
        compute_chunk(my)
        for r in range(N_RING):
            stage[r, 0] = out_ref[rows(my), cols(r)].astype(jnp.bfloat16)
        pl.semaphore_wait(barrier_sem, 2)
        pending = [rs_rdma(r, 0) for r in range(N_RING)]
        for p in pending:
            p.start()
        for d in (1, 7, 2, 6, 3, 5, 4):
            compute_chunk(mod8(my + d))

        for s in range(N_DEV - 2):
            sl = s % 2
            for grp in GROUPS:
                for r in grp:
                    pending[r].wait()
                for r in grp:
                    c = c_recv_rs(r, s)
                    t = (out_ref[rows(c), cols(r)]
                         + rs_recv[r, sl].astype(jnp.float32))
                    stage[r, (s + 1) % 2] = t.astype(jnp.bfloat16)
                    if s <= N_DEV - 4:
                        signal(rs_cr[r], writer(r))
                for r in grp:
                    if s >= 1:
                        pl.semaphore_wait(rs_cr[r], 1)
                    nxt = rs_rdma(r, s + 1)
                    nxt.start()
                    pending[r] = nxt

        for grp in GROUPS:
            for r in grp:
                pending[r].wait()
            ts = {}
            for r in grp:
                c = c_recv_rs(r, N_DEV - 2)
                t = (out_ref[rows(c), cols(r)]
                     + rs_recv[r, (N_DEV - 2) % 2].astype(jnp.float32))
                stage[r, 0] = t.astype(jnp.bfloat16)
                ts[r] = (c, t)
            for r in grp:
                nxt = ag_rdma(r, 0)
                nxt.start()
                pending[r] = nxt
            for r in grp:
                c, t = ts[r]
                out_ref[rows(c), cols(r)] = t

        for s in range(N_DEV - 1):
            sl = s % 2
            for grp in GROUPS:
                for r in grp:
                    pending[r].wait()
                for r in grp:
                    if 1 <= s <= N_DEV - 3:
                        signal(ag_cr[r], writer(r))
                if s < N_DEV - 2:
                    for r in grp:
                        if s >= 1:
                            pl.semaphore_wait(ag_cr[r], 1)
                        nxt = ag_rdma(r, s + 1)
                        nxt.start()
                        pending[r] = nxt
                for r in grp:
                    out_ref[rows(c_recv_ag(r, s)), cols(r)] = (
                        ag_buf[r, sl].astype(jnp.float32))

    n_sems = 4 * N_RING
    comm = (N_RING, 2, chunk, nq)
    out = pl.pallas_call(
        body,
        out_shape=jax.ShapeDtypeStruct((m, n), jnp.float32),
        in_specs=[
            pl.BlockSpec(memory_space=pltpu.VMEM),
            pl.BlockSpec(memory_space=pltpu.VMEM),
        ],
        out_specs=pl.BlockSpec(memory_space=pltpu.VMEM),
        scratch_shapes=[
            pltpu.VMEM((k, n), jnp.bfloat16),
            pltpu.VMEM(comm, jnp.bfloat16),
            pltpu.VMEM(comm, jnp.bfloat16),
            pltpu.VMEM(comm, jnp.bfloat16),
            pltpu.SemaphoreType.DMA((n_sems,)),
            pltpu.SemaphoreType.DMA((n_sems,)),
        ] + [pltpu.SemaphoreType.REGULAR] * (2 * N_RING),
        compiler_params=pltpu.CompilerParams(
            collective_id=0,
            vmem_limit_bytes=63 * 1024 * 1024,
        ),
    )(x, w_mat)
    return out
